# baseline (device time: 215819 ns/iter reference)
import jax
import jax.numpy as jnp
from jax import lax
from jax.experimental import pallas as pl
from jax.experimental.pallas import tpu as pltpu

T = 2048
D = 4096
V_SHARD = 8192
V_BLK = 512
NV = V_SHARD // V_BLK

_DeviceIdType = getattr(pl, "DeviceIdType", None) or pltpu.DeviceIdType
_sem_signal = getattr(pl, "semaphore_signal", None) or pltpu.semaphore_signal
_sem_wait = getattr(pl, "semaphore_wait", None) or pltpu.semaphore_wait
_CompilerParams = getattr(pltpu, "CompilerParams", None) or pltpu.TPUCompilerParams


def kernel(x, W, labels):
    xb = x.astype(jnp.bfloat16)
    lab2 = labels.reshape(T, 1)

    def body(x_ref, w_ref, lab_ref, out_ref,
             lga_ref, lgb_ref, s_ref, ll_ref, send_ref, recv_ref,
             send_sem, recv_sem):
        j = pl.program_id(0)
        my_x = lax.axis_index("x")
        my_y = lax.axis_index("y")
        my_z = lax.axis_index("z")
        neighbor = (my_x, 1 - my_y, my_z)

        @pl.when(j == 0)
        def _init():
            barrier = pltpu.get_barrier_semaphore()
            _sem_signal(barrier, inc=1, device_id=neighbor,
                        device_id_type=_DeviceIdType.MESH)
            _sem_wait(barrier, 1)
            s_ref[...] = jnp.zeros_like(s_ref)
            ll_ref[...] = jnp.zeros_like(ll_ref)

        ones = jnp.ones((V_BLK, 1), jnp.bfloat16)
        iota = lax.broadcasted_iota(jnp.int32, (T, V_BLK), 1)

        def epilogue(jj, lg_ref):
            lg = lg_ref[...]
            e = lg
            s_ref[...] += jnp.dot(e, ones, preferred_element_type=jnp.float32)
            diff = lab_ref[...] - (my_y * V_SHARD + jj * V_BLK)
            masked = jnp.where(iota == diff, lg, jnp.bfloat16(0.0))
            ll_ref[...] += jnp.dot(masked, ones,
                                   preferred_element_type=jnp.float32)

        def matmul(dst_ref):
            dst_ref[...] = jnp.dot(
                x_ref[...], w_ref[...].astype(jnp.bfloat16),
                preferred_element_type=jnp.float32).astype(jnp.bfloat16)

        @pl.when(lax.rem(j, 2) == 0)
        def _even():
            @pl.when(j > 0)
            def _prev():
                epilogue(j - 1, lgb_ref)
            matmul(lga_ref)

        @pl.when(lax.rem(j, 2) == 1)
        def _odd():
            epilogue(j - 1, lga_ref)
            matmul(lgb_ref)

        @pl.when(j == NV - 1)
        def _finish():
            epilogue(j, lgb_ref if (NV - 1) % 2 == 1 else lga_ref)
            send_ref[:, 0:1] = s_ref[...]
            send_ref[:, 1:2] = ll_ref[...]
            send_ref[:, 2:3] = s_ref[...]
            send_ref[:, 3:4] = ll_ref[...]
            rdma = pltpu.make_async_remote_copy(
                src_ref=send_ref, dst_ref=recv_ref,
                send_sem=send_sem, recv_sem=recv_sem,
                device_id=neighbor, device_id_type=_DeviceIdType.MESH)
            rdma.start()
            rdma.wait()
            s_g = s_ref[...] + recv_ref[:, 0:1]
            ll_g = ll_ref[...] + recv_ref[:, 1:2]
            out_ref[...] = jnp.log(s_g) - ll_g

    out2 = pl.pallas_call(
        body,
        grid=(NV,),
        in_specs=[
            pl.BlockSpec(memory_space=pltpu.VMEM),
            pl.BlockSpec((D, V_BLK), lambda j: (0, j)),
            pl.BlockSpec(memory_space=pltpu.VMEM),
        ],
        out_specs=pl.BlockSpec(memory_space=pltpu.VMEM),
        out_shape=jax.ShapeDtypeStruct((T, 1), jnp.float32),
        scratch_shapes=[
            pltpu.VMEM((T, V_BLK), jnp.bfloat16),
            pltpu.VMEM((T, V_BLK), jnp.bfloat16),
            pltpu.VMEM((T, 1), jnp.float32),
            pltpu.VMEM((T, 1), jnp.float32),
            pltpu.VMEM((T, 4), jnp.float32),
            pltpu.VMEM((T, 4), jnp.float32),
            pltpu.SemaphoreType.DMA,
            pltpu.SemaphoreType.DMA,
        ],
        compiler_params=_CompilerParams(
            collective_id=0, vmem_limit_bytes=60 * 1024 * 1024),
    )(xb, W, lab2)
    return out2.reshape(T)


# device time: 204888 ns/iter; 1.0534x vs baseline; 1.0534x over previous
import jax
import jax.numpy as jnp
from jax import lax
from jax.experimental import pallas as pl
from jax.experimental.pallas import tpu as pltpu

T = 2048
D = 4096
V_SHARD = 8192
V_BLK = 512
NV = V_SHARD // V_BLK

_DeviceIdType = getattr(pl, "DeviceIdType", None) or pltpu.DeviceIdType
_sem_signal = getattr(pl, "semaphore_signal", None) or pltpu.semaphore_signal
_sem_wait = getattr(pl, "semaphore_wait", None) or pltpu.semaphore_wait
_CompilerParams = getattr(pltpu, "CompilerParams", None) or pltpu.TPUCompilerParams


def kernel(x, W, labels):
    xb = x.astype(jnp.bfloat16)
    lab2 = labels.reshape(T, 1)

    def body(x_ref, w_ref, lab_ref, out_ref,
             lga_ref, lgb_ref, eacc_ref, llacc_ref, send_ref, recv_ref,
             send_sem, recv_sem):
        j = pl.program_id(0)
        my_x = lax.axis_index("x")
        my_y = lax.axis_index("y")
        my_z = lax.axis_index("z")
        neighbor = (my_x, 1 - my_y, my_z)

        @pl.when(j == 0)
        def _init():
            barrier = pltpu.get_barrier_semaphore()
            _sem_signal(barrier, inc=1, device_id=neighbor,
                        device_id_type=_DeviceIdType.MESH)
            _sem_wait(barrier, 1)
            eacc_ref[...] = jnp.zeros_like(eacc_ref)
            llacc_ref[...] = jnp.zeros_like(llacc_ref)

        iota = lax.broadcasted_iota(jnp.int32, (T, V_BLK), 1)

        def epilogue(jj, lg_ref):
            lg = lg_ref[...]
            eacc_ref[...] += jnp.exp(lg)
            diff = lab_ref[...] - (my_y * V_SHARD + jj * V_BLK)
            llacc_ref[...] += jnp.where(iota == diff, lg, jnp.bfloat16(0.0))

        def matmul(dst_ref):
            dst_ref[...] = jnp.dot(
                x_ref[...], w_ref[...].astype(jnp.bfloat16),
                preferred_element_type=jnp.float32).astype(jnp.bfloat16)

        @pl.when(lax.rem(j, 2) == 0)
        def _even():
            @pl.when(j > 0)
            def _prev():
                epilogue(j - 1, lgb_ref)
            matmul(lga_ref)

        @pl.when(lax.rem(j, 2) == 1)
        def _odd():
            epilogue(j - 1, lga_ref)
            matmul(lgb_ref)

        @pl.when(j == NV - 1)
        def _finish():
            epilogue(j, lgb_ref if (NV - 1) % 2 == 1 else lga_ref)
            ones = jnp.ones((V_BLK, 1), jnp.bfloat16)
            s_l = jnp.dot(eacc_ref[...], ones,
                          preferred_element_type=jnp.float32)
            ll_l = jnp.dot(llacc_ref[...], ones,
                           preferred_element_type=jnp.float32)
            send_ref[:, 0:1] = s_l
            send_ref[:, 1:2] = ll_l
            send_ref[:, 2:3] = s_l
            send_ref[:, 3:4] = ll_l
            rdma = pltpu.make_async_remote_copy(
                src_ref=send_ref, dst_ref=recv_ref,
                send_sem=send_sem, recv_sem=recv_sem,
                device_id=neighbor, device_id_type=_DeviceIdType.MESH)
            rdma.start()
            rdma.wait()
            s_g = s_l + recv_ref[:, 0:1]
            ll_g = ll_l + recv_ref[:, 1:2]
            out_ref[...] = jnp.log(s_g) - ll_g

    out2 = pl.pallas_call(
        body,
        grid=(NV,),
        in_specs=[
            pl.BlockSpec(memory_space=pltpu.VMEM),
            pl.BlockSpec((D, V_BLK), lambda j: (0, j)),
            pl.BlockSpec(memory_space=pltpu.VMEM),
        ],
        out_specs=pl.BlockSpec(memory_space=pltpu.VMEM),
        out_shape=jax.ShapeDtypeStruct((T, 1), jnp.float32),
        scratch_shapes=[
            pltpu.VMEM((T, V_BLK), jnp.bfloat16),
            pltpu.VMEM((T, V_BLK), jnp.bfloat16),
            pltpu.VMEM((T, V_BLK), jnp.bfloat16),
            pltpu.VMEM((T, V_BLK), jnp.bfloat16),
            pltpu.VMEM((T, 4), jnp.float32),
            pltpu.VMEM((T, 4), jnp.float32),
            pltpu.SemaphoreType.DMA,
            pltpu.SemaphoreType.DMA,
        ],
        compiler_params=_CompilerParams(
            collective_id=0, vmem_limit_bytes=60 * 1024 * 1024),
    )(xb, W, lab2)
    return out2.reshape(T)


# device time: 201987 ns/iter; 1.0685x vs baseline; 1.0144x over previous
import jax
import jax.numpy as jnp
from jax import lax
from jax.experimental import pallas as pl
from jax.experimental.pallas import tpu as pltpu

T = 2048
D = 4096
V_SHARD = 8192
V_BLK = 512
NV = V_SHARD // V_BLK

_DeviceIdType = getattr(pl, "DeviceIdType", None) or pltpu.DeviceIdType
_sem_signal = getattr(pl, "semaphore_signal", None) or pltpu.semaphore_signal
_sem_wait = getattr(pl, "semaphore_wait", None) or pltpu.semaphore_wait
_CompilerParams = getattr(pltpu, "CompilerParams", None) or pltpu.TPUCompilerParams


def kernel(x, W, labels):
    xb = x.astype(jnp.bfloat16)
    lab2 = labels.reshape(T, 1)

    def body(x_ref, w_ref, lab_ref, out_ref,
             eacc_ref, llacc_ref, send_ref, recv_ref,
             send_sem, recv_sem):
        j = pl.program_id(0)
        my_x = lax.axis_index("x")
        my_y = lax.axis_index("y")
        my_z = lax.axis_index("z")
        neighbor = (my_x, 1 - my_y, my_z)

        @pl.when(j == 0)
        def _init():
            barrier = pltpu.get_barrier_semaphore()
            _sem_signal(barrier, inc=1, device_id=neighbor,
                        device_id_type=_DeviceIdType.MESH)
            _sem_wait(barrier, 1)
            eacc_ref[...] = jnp.zeros_like(eacc_ref)
            llacc_ref[...] = jnp.zeros_like(llacc_ref)

        lg = jnp.dot(x_ref[...], w_ref[...].astype(jnp.bfloat16),
                     preferred_element_type=jnp.float32).astype(jnp.bfloat16)
        eacc_ref[...] += jnp.exp(lg)
        iota16 = lax.broadcasted_iota(jnp.int16, (T, V_BLK), 1)
        diff16 = (lab_ref[...] - (my_y * V_SHARD + j * V_BLK)
                  ).astype(jnp.int16)
        llacc_ref[...] += jnp.where(iota16 == diff16, lg, jnp.bfloat16(0.0))

        @pl.when(j == NV - 1)
        def _finish():
            ones = jnp.ones((V_BLK, 1), jnp.bfloat16)
            s_l = jnp.dot(eacc_ref[...], ones,
                          preferred_element_type=jnp.float32)
            ll_l = jnp.dot(llacc_ref[...], ones,
                           preferred_element_type=jnp.float32)
            send_ref[:, 0:1] = s_l
            send_ref[:, 1:2] = ll_l
            send_ref[:, 2:3] = s_l
            send_ref[:, 3:4] = ll_l
            rdma = pltpu.make_async_remote_copy(
                src_ref=send_ref, dst_ref=recv_ref,
                send_sem=send_sem, recv_sem=recv_sem,
                device_id=neighbor, device_id_type=_DeviceIdType.MESH)
            rdma.start()
            rdma.wait()
            s_g = s_l + recv_ref[:, 0:1]
            ll_g = ll_l + recv_ref[:, 1:2]
            out_ref[...] = jnp.log(s_g) - ll_g

    out2 = pl.pallas_call(
        body,
        grid=(NV,),
        in_specs=[
            pl.BlockSpec(memory_space=pltpu.VMEM),
            pl.BlockSpec((D, V_BLK), lambda j: (0, j)),
            pl.BlockSpec(memory_space=pltpu.VMEM),
        ],
        out_specs=pl.BlockSpec(memory_space=pltpu.VMEM),
        out_shape=jax.ShapeDtypeStruct((T, 1), jnp.float32),
        scratch_shapes=[
            pltpu.VMEM((T, V_BLK), jnp.bfloat16),
            pltpu.VMEM((T, V_BLK), jnp.bfloat16),
            pltpu.VMEM((T, 4), jnp.float32),
            pltpu.VMEM((T, 4), jnp.float32),
            pltpu.SemaphoreType.DMA,
            pltpu.SemaphoreType.DMA,
        ],
        compiler_params=_CompilerParams(
            collective_id=0, vmem_limit_bytes=60 * 1024 * 1024),
    )(xb, W, lab2)
    return out2.reshape(T)


# device time: 123294 ns/iter; 1.7504x vs baseline; 1.6383x over previous
import jax
import jax.numpy as jnp
from jax import lax
from jax.experimental import pallas as pl
from jax.experimental.pallas import tpu as pltpu

T = 2048
D = 4096
V_SHARD = 8192
V_SUB = 1024
V_BLK = 512
NK = V_SUB // V_BLK
N_ROUNDS = 4

_DeviceIdType = getattr(pl, "DeviceIdType", None) or pltpu.DeviceIdType
_sem_signal = getattr(pl, "semaphore_signal", None) or pltpu.semaphore_signal
_sem_wait = getattr(pl, "semaphore_wait", None) or pltpu.semaphore_wait
_CompilerParams = getattr(pltpu, "CompilerParams", None) or pltpu.TPUCompilerParams


def kernel(x, W, labels):
    xb = x.astype(jnp.bfloat16)
    lab2 = labels.reshape(T, 1)

    def body(x_ref, w_ref, lab_ref, out_ref,
             wslice_ref, acc_ref, recv_ref,
             wdma_sem, send_sems, recv_sems):
        my_x = lax.axis_index("x")
        my_y = lax.axis_index("y")
        my_z = lax.axis_index("z")
        partners = [
            (my_x, my_y, jnp.bitwise_xor(my_z, 1)),
            (my_x, my_y, jnp.bitwise_xor(my_z, 2)),
            (1 - my_x, my_y, my_z),
            (my_x, 1 - my_y, my_z),
        ]

        barrier = pltpu.get_barrier_semaphore()
        for p in partners:
            _sem_signal(barrier, inc=1, device_id=p,
                        device_id_type=_DeviceIdType.MESH)
        _sem_wait(barrier, len(partners))

        sub = my_x * 4 + my_z
        col0 = sub * V_SUB
        wdma = pltpu.make_async_copy(
            w_ref.at[:, pl.ds(col0, V_SUB)], wslice_ref, wdma_sem)
        wdma.start()
        wdma.wait()

        iota = lax.broadcasted_iota(jnp.int32, (T, V_BLK), 1)
        s2d = jnp.zeros((T, 1), jnp.float32)
        ll2d = jnp.zeros((T, 1), jnp.float32)
        for k in range(NK):
            wb = wslice_ref[:, k * V_BLK:(k + 1) * V_BLK].astype(jnp.bfloat16)
            lg = jnp.dot(x_ref[...], wb, preferred_element_type=jnp.float32)
            s2d += jnp.sum(jnp.exp(lg), axis=1, keepdims=True)
            diff = lab_ref[...] - (my_y * V_SHARD + col0 + k * V_BLK)
            ll2d += jnp.sum(jnp.where(iota == diff, lg, 0.0),
                            axis=1, keepdims=True)

        acc_ref[:, 0:1] = s2d
        acc_ref[:, 1:2] = ll2d

        for r, p in enumerate(partners):
            rdma = pltpu.make_async_remote_copy(
                src_ref=acc_ref, dst_ref=recv_ref.at[r],
                send_sem=send_sems.at[r], recv_sem=recv_sems.at[r],
                device_id=p, device_id_type=_DeviceIdType.MESH)
            rdma.start()
            rdma.wait()
            acc_ref[:, 0:2] += recv_ref[r, :, 0:2]

        out_ref[...] = jnp.log(acc_ref[:, 0:1]) - acc_ref[:, 1:2]

    out2 = pl.pallas_call(
        body,
        in_specs=[
            pl.BlockSpec(memory_space=pltpu.VMEM),
            pl.BlockSpec(memory_space=pl.ANY),
            pl.BlockSpec(memory_space=pltpu.VMEM),
        ],
        out_specs=pl.BlockSpec(memory_space=pltpu.VMEM),
        out_shape=jax.ShapeDtypeStruct((T, 1), jnp.float32),
        scratch_shapes=[
            pltpu.VMEM((D, V_SUB), jnp.float32),
            pltpu.VMEM((T, 8), jnp.float32),
            pltpu.VMEM((N_ROUNDS, T, 8), jnp.float32),
            pltpu.SemaphoreType.DMA,
            pltpu.SemaphoreType.DMA((N_ROUNDS,)),
            pltpu.SemaphoreType.DMA((N_ROUNDS,)),
        ],
        compiler_params=_CompilerParams(
            collective_id=0, vmem_limit_bytes=60 * 1024 * 1024),
    )(xb, W, lab2)
    return out2.reshape(T)


# device time: 73472 ns/iter; 2.9374x vs baseline; 1.6781x over previous
import jax
import jax.numpy as jnp
from jax import lax
from jax.experimental import pallas as pl
from jax.experimental.pallas import tpu as pltpu

T = 2048
D = 4096
V_SHARD = 8192
V_SUB = 1024
V_BLK = 512
NK = V_SUB // V_BLK
N_ROUNDS = 4

_DeviceIdType = getattr(pl, "DeviceIdType", None) or pltpu.DeviceIdType
_sem_signal = getattr(pl, "semaphore_signal", None) or pltpu.semaphore_signal
_sem_wait = getattr(pl, "semaphore_wait", None) or pltpu.semaphore_wait
_CompilerParams = getattr(pltpu, "CompilerParams", None) or pltpu.TPUCompilerParams


def kernel(x, W, labels):
    xb = x.astype(jnp.bfloat16)
    lab2 = labels.reshape(T, 1)

    def body(x_ref, w_ref, lab_ref, out_ref,
             wslice_ref, acc_ref, recv_ref,
             wdma_sem, send_sems, recv_sems):
        my_x = lax.axis_index("x")
        my_y = lax.axis_index("y")
        my_z = lax.axis_index("z")
        partners = [
            (my_x, my_y, jnp.bitwise_xor(my_z, 1)),
            (my_x, my_y, jnp.bitwise_xor(my_z, 2)),
            (1 - my_x, my_y, my_z),
            (my_x, 1 - my_y, my_z),
        ]

        barrier = pltpu.get_barrier_semaphore()
        for p in partners:
            _sem_signal(barrier, inc=1, device_id=p,
                        device_id_type=_DeviceIdType.MESH)
        _sem_wait(barrier, len(partners))

        sub = my_x * 4 + my_z
        col0 = sub * V_SUB
        wdma = pltpu.make_async_copy(
            w_ref.at[:, pl.ds(col0, V_SUB)], wslice_ref, wdma_sem)
        wdma.start()
        wdma.wait()

        iota = lax.broadcasted_iota(jnp.int32, (T, V_BLK), 1)
        s1d = jnp.zeros((T,), jnp.float32)
        ll1d = jnp.zeros((T,), jnp.float32)
        for k in range(NK):
            wb = wslice_ref[:, k * V_BLK:(k + 1) * V_BLK].astype(jnp.bfloat16)
            lg = jnp.dot(x_ref[...], wb, preferred_element_type=jnp.float32)
            s1d += jnp.sum(jnp.exp(lg), axis=1)
            diff = lab_ref[...] - (my_y * V_SHARD + col0 + k * V_BLK)
            ll1d += jnp.sum(jnp.where(iota == diff, lg, 0.0), axis=1)

        acc_ref[0, :] = s1d
        acc_ref[1, :] = ll1d

        for r, p in enumerate(partners):
            rdma = pltpu.make_async_remote_copy(
                src_ref=acc_ref, dst_ref=recv_ref.at[r],
                send_sem=send_sems.at[r], recv_sem=recv_sems.at[r],
                device_id=p, device_id_type=_DeviceIdType.MESH)
            rdma.start()
            rdma.wait()
            acc_ref[0:2, :] += recv_ref[r, 0:2, :]

        out_ref[...] = jnp.log(acc_ref[0, :]) - acc_ref[1, :]

    out2 = pl.pallas_call(
        body,
        in_specs=[
            pl.BlockSpec(memory_space=pltpu.VMEM),
            pl.BlockSpec(memory_space=pl.ANY),
            pl.BlockSpec(memory_space=pltpu.VMEM),
        ],
        out_specs=pl.BlockSpec(memory_space=pltpu.VMEM),
        out_shape=jax.ShapeDtypeStruct((T,), jnp.float32),
        scratch_shapes=[
            pltpu.VMEM((D, V_SUB), jnp.float32),
            pltpu.VMEM((8, T), jnp.float32),
            pltpu.VMEM((N_ROUNDS, 8, T), jnp.float32),
            pltpu.SemaphoreType.DMA,
            pltpu.SemaphoreType.DMA((N_ROUNDS,)),
            pltpu.SemaphoreType.DMA((N_ROUNDS,)),
        ],
        compiler_params=_CompilerParams(
            collective_id=0, vmem_limit_bytes=60 * 1024 * 1024),
    )(xb, W, lab2)
    return out2


# device time: 61941 ns/iter; 3.4843x vs baseline; 1.1862x over previous
import jax
import jax.numpy as jnp
from jax import lax
from jax.experimental import pallas as pl
from jax.experimental.pallas import tpu as pltpu

T = 2048
D = 4096
V_SHARD = 8192
V_SUB = 1024
V_BLK = 512
NK = V_SUB // V_BLK
T_CHUNK = 256
NC = T // T_CHUNK
N_ROUNDS = 4

_DeviceIdType = getattr(pl, "DeviceIdType", None) or pltpu.DeviceIdType
_sem_signal = getattr(pl, "semaphore_signal", None) or pltpu.semaphore_signal
_sem_wait = getattr(pl, "semaphore_wait", None) or pltpu.semaphore_wait
_CompilerParams = getattr(pltpu, "CompilerParams", None) or pltpu.TPUCompilerParams


def kernel(x, W, labels):
    lab2 = labels.reshape(T, 1)

    def body(x_ref, w_ref, lab_ref, out_ref,
             xb_ref, xstage_ref, wslice_ref, acc_ref, recv_ref,
             xdma_sems, wdma_sem, send_sems, recv_sems):
        my_x = lax.axis_index("x")
        my_y = lax.axis_index("y")
        my_z = lax.axis_index("z")
        partners = [
            (my_x, my_y, jnp.bitwise_xor(my_z, 1)),
            (my_x, my_y, jnp.bitwise_xor(my_z, 2)),
            (1 - my_x, my_y, my_z),
            (my_x, 1 - my_y, my_z),
        ]

        barrier = pltpu.get_barrier_semaphore()
        for p in partners:
            _sem_signal(barrier, inc=1, device_id=p,
                        device_id_type=_DeviceIdType.MESH)
        _sem_wait(barrier, N_ROUNDS)

        sub = my_x * 4 + my_z
        col0 = sub * V_SUB
        wdma = pltpu.make_async_copy(
            w_ref.at[:, pl.ds(col0, V_SUB)], wslice_ref, wdma_sem)
        wdma.start()

        cps = [
            pltpu.make_async_copy(
                x_ref.at[pl.ds(c * T_CHUNK, T_CHUNK), :],
                xstage_ref.at[c % 2], xdma_sems.at[c % 2])
            for c in range(NC)
        ]
        cps[0].start()
        cps[1].start()
        for c in range(NC):
            cps[c].wait()
            xb_ref[pl.ds(c * T_CHUNK, T_CHUNK), :] = (
                xstage_ref[c % 2].astype(jnp.bfloat16))
            if c + 2 < NC:
                cps[c + 2].start()

        wdma.wait()

        iota = lax.broadcasted_iota(jnp.int32, (T, V_BLK), 1)
        s1d = jnp.zeros((T,), jnp.float32)
        ll1d = jnp.zeros((T,), jnp.float32)
        for k in range(NK):
            wb = wslice_ref[:, k * V_BLK:(k + 1) * V_BLK].astype(jnp.bfloat16)
            lg = jnp.dot(xb_ref[...], wb, preferred_element_type=jnp.float32)
            s1d += jnp.sum(jnp.exp(lg), axis=1)
            diff = lab_ref[...] - (my_y * V_SHARD + col0 + k * V_BLK)
            ll1d += jnp.sum(jnp.where(iota == diff, lg, 0.0), axis=1)

        acc_ref[0, :] = s1d
        acc_ref[1, :] = ll1d

        for r, p in enumerate(partners):
            rdma = pltpu.make_async_remote_copy(
                src_ref=acc_ref, dst_ref=recv_ref.at[r],
                send_sem=send_sems.at[r], recv_sem=recv_sems.at[r],
                device_id=p, device_id_type=_DeviceIdType.MESH)
            rdma.start()
            rdma.wait()
            acc_ref[0:2, :] += recv_ref[r, 0:2, :]

        out_ref[...] = jnp.log(acc_ref[0, :]) - acc_ref[1, :]

    out2 = pl.pallas_call(
        body,
        in_specs=[
            pl.BlockSpec(memory_space=pl.ANY),
            pl.BlockSpec(memory_space=pl.ANY),
            pl.BlockSpec(memory_space=pltpu.VMEM),
        ],
        out_specs=pl.BlockSpec(memory_space=pltpu.VMEM),
        out_shape=jax.ShapeDtypeStruct((T,), jnp.float32),
        scratch_shapes=[
            pltpu.VMEM((T, D), jnp.bfloat16),
            pltpu.VMEM((2, T_CHUNK, D), jnp.float32),
            pltpu.VMEM((D, V_SUB), jnp.float32),
            pltpu.VMEM((8, T), jnp.float32),
            pltpu.VMEM((N_ROUNDS, 8, T), jnp.float32),
            pltpu.SemaphoreType.DMA((2,)),
            pltpu.SemaphoreType.DMA,
            pltpu.SemaphoreType.DMA((N_ROUNDS,)),
            pltpu.SemaphoreType.DMA((N_ROUNDS,)),
        ],
        compiler_params=_CompilerParams(
            collective_id=0, vmem_limit_bytes=62 * 1024 * 1024),
    )(x, W, lab2)
    return out2
